# Initial kernel scaffold; baseline (speedup 1.0000x reference)
import jax
import jax.numpy as jnp
from jax.experimental import pallas as pl


def kernel(tokens, table):
    raise NotImplementedError("write your pallas kernel here")


# speedup vs baseline: 2.2930x; 2.2930x over previous
import math

import jax
import jax.numpy as jnp
from jax import lax
from jax.experimental import pallas as pl
from jax.experimental.pallas import tpu as pltpu
from jax.experimental.pallas import tpu_sc as plsc

VOCAB = 100000
EMB = 128
SCALE = math.sqrt(EMB)

NC, NS, L = 2, 16, 16
NW = NC * NS

B = 4096 * 50
B_PER_W = B // NW
CHUNK = 128
N_CHUNKS = B_PER_W // CHUNK


def _emb_body(tok_hbm, table_hbm, out_hbm, idx_v, rows_v, gsem):
    wid = lax.axis_index("s") * NC + lax.axis_index("c")
    base = wid * B_PER_W

    def chunk(j, carry):
        off = base + j * CHUNK
        pltpu.sync_copy(tok_hbm.at[pl.ds(off, CHUNK)], idx_v)
        pltpu.async_copy(table_hbm.at[idx_v], rows_v, gsem).wait()

        def srow(i, c2):
            for c in range(EMB // L):
                sl = (i, pl.ds(c * L, L))
                rows_v[sl] = rows_v[sl] * SCALE
            return c2

        lax.fori_loop(0, CHUNK, srow, 0)
        pltpu.sync_copy(rows_v, out_hbm.at[pl.ds(off, CHUNK)])
        return carry

    lax.fori_loop(0, N_CHUNKS, chunk, 0)


@jax.jit
def _emb(tokens_flat, table):
    mesh = plsc.VectorSubcoreMesh(core_axis_name="c", subcore_axis_name="s")
    f = pl.kernel(
        _emb_body,
        out_type=jax.ShapeDtypeStruct((B, EMB), jnp.float32),
        mesh=mesh,
        scratch_types=[
            pltpu.VMEM((CHUNK,), jnp.int32),
            pltpu.VMEM((CHUNK, EMB), jnp.float32),
            pltpu.SemaphoreType.DMA,
        ],
    )
    return f(tokens_flat, table)


def kernel(tokens, table):
    out = _emb(tokens.reshape(-1).astype(jnp.int32), table)
    return out.reshape(tokens.shape + (EMB,))


# speedup vs baseline: 2.9023x; 1.2657x over previous
import math

import jax
import jax.numpy as jnp
from jax import lax
from jax.experimental import pallas as pl
from jax.experimental.pallas import tpu as pltpu
from jax.experimental.pallas import tpu_sc as plsc

VOCAB = 100000
EMB = 128
SCALE = math.sqrt(EMB)

NC, NS, L = 2, 16, 16
NW = NC * NS

B = 4096 * 50
B_PER_W = B // NW
CHUNK = 128
N_CHUNKS = B_PER_W // CHUNK
NBUF = 3


def _emb_body(tok_hbm, table_hbm, out_hbm, idx_all, rows_v,
              g0, g1, g2, s0, s1, s2):
    gsem = [g0, g1, g2]
    ssem = [s0, s1, s2]
    wid = lax.axis_index("s") * NC + lax.axis_index("c")
    base = wid * B_PER_W

    pltpu.sync_copy(tok_hbm.at[wid], idx_all)

    def gather(j, b):
        return pltpu.async_copy(table_hbm.at[idx_all.at[j]], rows_v.at[b],
                                gsem[b])

    def store(j, b):
        return pltpu.async_copy(rows_v.at[b],
                                out_hbm.at[pl.ds(base + j * CHUNK, CHUNK)],
                                ssem[b])

    def scale(b):
        def srow(i, c2):
            for c in range(EMB // L):
                sl = (b, i, pl.ds(c * L, L))
                rows_v[sl] = rows_v[sl] * SCALE
            return c2
        lax.fori_loop(0, CHUNK, srow, 0)

    gd, sd = {}, {}
    gd[0] = gather(0, 0)
    gd[1] = gather(1, 1)
    for j in range(N_CHUNKS):
        b = j % NBUF
        if j + 2 < N_CHUNKS:
            if j - 1 >= 0:
                sd[j - 1].wait()
            gd[j + 2] = gather(j + 2, (j + 2) % NBUF)
        gd[j].wait()
        scale(b)
        sd[j] = store(j, b)
    for j in range(max(0, N_CHUNKS - NBUF), N_CHUNKS):
        sd[j].wait()


@jax.jit
def _emb(tokens_grid, table):
    mesh = plsc.VectorSubcoreMesh(core_axis_name="c", subcore_axis_name="s")
    f = pl.kernel(
        _emb_body,
        out_type=jax.ShapeDtypeStruct((B, EMB), jnp.float32),
        mesh=mesh,
        scratch_types=[
            pltpu.VMEM((N_CHUNKS, CHUNK), jnp.int32),
            pltpu.VMEM((NBUF, CHUNK, EMB), jnp.float32),
            pltpu.SemaphoreType.DMA,
            pltpu.SemaphoreType.DMA,
            pltpu.SemaphoreType.DMA,
            pltpu.SemaphoreType.DMA,
            pltpu.SemaphoreType.DMA,
            pltpu.SemaphoreType.DMA,
        ],
    )
    return f(tokens_grid, table)


def kernel(tokens, table):
    tok = tokens.reshape(NW, N_CHUNKS, CHUNK).astype(jnp.int32)
    out = _emb(tok, table)
    return out.reshape(tokens.shape + (EMB,))
